# Initial kernel scaffold; baseline (speedup 1.0000x reference)
import jax
import jax.numpy as jnp
from jax.experimental import pallas as pl


def kernel(x, edge_index, edge_type, lin_w, lin_b, comp1, bases1, root1, bias1, comp2, bases2, root2, bias2):
    raise NotImplementedError("write your pallas kernel here")


# speedup vs baseline: 5.9481x; 5.9481x over previous
import functools

import jax
import jax.numpy as jnp
from jax import lax
from jax.experimental import pallas as pl
from jax.experimental.pallas import tpu as pltpu
from jax.experimental.pallas import tpu_sc as plsc

N = 10000
E = 320000
R = 5
NB = 30
D = 128

NC, NS = 2, 16
NBKT = 4
BUCKET = 2560
NSEG = R * BUCKET
TRASH = NSEG
ACC_ROWS = NSEG + 16
PT = BUCKET // NS
EC = E // NS
STRIP = 2000
NSTRIP = EC // STRIP
CH = 80
NCH = STRIP // CH
ZB = 32


@functools.lru_cache(maxsize=None)
def _make_sc_segsum(with_cnt: bool):
    mesh = plsc.VectorSubcoreMesh(
        core_axis_name="c", subcore_axis_name="s",
        num_cores=NC, num_subcores=NS)
    out_type = [jax.ShapeDtypeStruct((NBKT, R, BUCKET, D), jnp.float32)]
    if with_cnt:
        out_type.append(jax.ShapeDtypeStruct((NBKT * R * BUCKET,), jnp.float32))
    scratch = [
        pltpu.VMEM((STRIP,), jnp.int32),
        pltpu.VMEM((STRIP,), jnp.int32),
        pltpu.VMEM((STRIP,), jnp.int32),
        pltpu.VMEM((1, CH), jnp.int32),
        pltpu.VMEM((CH, D), jnp.float32),
        pltpu.VMEM((ZB, D), jnp.float32),
        pltpu.VMEM((CH,), jnp.float32),
        pltpu.VMEM_SHARED((ACC_ROWS, D), jnp.float32),
        pltpu.VMEM_SHARED((ACC_ROWS,), jnp.float32),
    ]

    def body(h, srcr, dstr, etr, *rest):
        if with_cnt:
            (s_out, cnt_out, src_s, dst_s, et_s, seg2d,
             rows, zb, ones, acc, cnt_acc) = rest
        else:
            (s_out, src_s, dst_s, et_s, seg2d,
             rows, zb, ones, acc, cnt_acc) = rest
        c = lax.axis_index("c")
        t = lax.axis_index("s")

        zv = jnp.zeros((16,), jnp.float32)

        def zrow(i, carry):
            for k in range(D // 16):
                zb[i, pl.ds(k * 16, 16)] = zv
            return carry

        lax.fori_loop(0, ZB, zrow, 0)
        if with_cnt:
            ov = jnp.ones((16,), jnp.float32)
            for k in range(CH // 16):
                ones[pl.ds(k * 16, 16)] = ov

        tr_iota = TRASH + lax.iota(jnp.int32, 16)

        for qi in range(NBKT // NC):
            q = c * (NBKT // NC) + qi
            lo = q * BUCKET
            hi = lo + BUCKET

            for r in range(R):
                base = r * BUCKET + t * PT
                for z in range(PT // ZB):
                    pltpu.sync_copy(zb, acc.at[pl.ds(base + z * ZB, ZB)])
                if with_cnt:
                    pltpu.sync_copy(zb.at[0, pl.ds(0, D)],
                                    cnt_acc.at[pl.ds(base, D)])
                    pltpu.sync_copy(zb.at[0, pl.ds(0, PT - D)],
                                    cnt_acc.at[pl.ds(base + D, PT - D)])
            @pl.when(t == 0)
            def _zero_trash():
                pltpu.sync_copy(zb.at[pl.ds(0, 16)], acc.at[pl.ds(NSEG, 16)])
                if with_cnt:
                    pltpu.sync_copy(zb.at[0, pl.ds(0, 16)],
                                    cnt_acc.at[pl.ds(NSEG, 16)])
            plsc.subcore_barrier()

            for si in range(NSTRIP):
                ebase = t * EC + si * STRIP
                pltpu.sync_copy(srcr.at[pl.ds(ebase, STRIP)], src_s)
                pltpu.sync_copy(dstr.at[pl.ds(ebase, STRIP)], dst_s)
                pltpu.sync_copy(etr.at[pl.ds(ebase, STRIP)], et_s)

                def chunk_body(k, carry):
                    for g in range(CH // 16):
                        dv = dst_s[pl.ds(k * CH + g * 16, 16)]
                        ev = et_s[pl.ds(k * CH + g * 16, 16)]
                        m = (dv >= lo) & (dv < hi)
                        seg = jnp.where(m, ev * BUCKET + (dv - lo), tr_iota)
                        seg2d[0, pl.ds(g * 16, 16)] = seg
                    pltpu.sync_copy(h.at[src_s.at[pl.ds(k * CH, CH)]], rows)
                    pltpu.sync_copy(rows, acc.at[seg2d.at[0]], add=True)
                    if with_cnt:
                        pltpu.sync_copy(ones, cnt_acc.at[seg2d.at[0]],
                                        add=True)
                    return carry

                lax.fori_loop(0, NCH, chunk_body, 0)

            plsc.subcore_barrier()

            for r in range(R):
                base = r * BUCKET + t * PT
                pltpu.sync_copy(acc.at[pl.ds(base, PT)],
                                s_out.at[q, r, pl.ds(t * PT, PT)])
            if with_cnt:
                for j in range(NSEG // 128 // NS + 1):
                    cid = t + NS * j

                    @pl.when(cid < NSEG // 128)
                    def _dump_cnt():
                        pltpu.sync_copy(
                            cnt_acc.at[pl.ds(cid * 128, 128)],
                            cnt_out.at[pl.ds(q * NSEG + cid * 128, 128)])
            plsc.subcore_barrier()

    return pl.kernel(body, out_type=tuple(out_type), mesh=mesh,
                     compiler_params=pltpu.CompilerParams(
                         needs_layout_passes=False),
                     scratch_types=tuple(scratch))


def _lin_body(x_ref, w_ref, b_ref, o_ref):
    o_ref[...] = (jnp.dot(x_ref[...], w_ref[...],
                          preferred_element_type=jnp.float32) + b_ref[...])


def _tc_linear(x, w, b):
    bm = 2000
    return pl.pallas_call(
        _lin_body,
        grid=(N // bm,),
        in_specs=[pl.BlockSpec((bm, D), lambda i: (i, 0)),
                  pl.BlockSpec((D, D), lambda i: (0, 0)),
                  pl.BlockSpec((1, D), lambda i: (0, 0))],
        out_specs=pl.BlockSpec((bm, D), lambda i: (i, 0)),
        out_shape=jax.ShapeDtypeStruct((N, D), jnp.float32),
    )(x, w, b)


def _basis_body(c_ref, b_ref, o_ref):
    o_ref[...] = jnp.dot(c_ref[...], b_ref[...],
                         preferred_element_type=jnp.float32)


def _tc_basis(comp, bases):
    comp_pad = jnp.zeros((8, NB), jnp.float32).at[:R].set(comp)
    bases_flat = bases.reshape(NB, D * D)
    w = pl.pallas_call(
        _basis_body,
        in_specs=[pl.BlockSpec((8, NB), lambda: (0, 0)),
                  pl.BlockSpec((NB, D * D), lambda: (0, 0))],
        out_specs=pl.BlockSpec((8, D * D), lambda: (0, 0)),
        out_shape=jax.ShapeDtypeStruct((8, D * D), jnp.float32),
    )(comp_pad, bases_flat)
    return w[:R].reshape(R, D, D)


def _combine_body(h_ref, root_ref, b_ref, s_ref, c_ref, w_ref, o_ref, *,
                  softmax):
    out = (jnp.dot(h_ref[...], root_ref[...],
                   preferred_element_type=jnp.float32) + b_ref[...])
    for r in range(R):
        mean = s_ref[0, r] / jnp.maximum(c_ref[0, r], 1.0)
        out = out + jnp.dot(mean, w_ref[r], preferred_element_type=jnp.float32)
    if softmax:
        m = jnp.max(out, axis=1, keepdims=True)
        e = jnp.exp(out - m)
        out = e / jnp.sum(e, axis=1, keepdims=True)
    o_ref[...] = out


def _tc_combine(h, root, b, s, cnt4, w, softmax):
    return pl.pallas_call(
        functools.partial(_combine_body, softmax=softmax),
        grid=(NBKT,),
        in_specs=[pl.BlockSpec((BUCKET, D), lambda q: (q, 0)),
                  pl.BlockSpec((D, D), lambda q: (0, 0)),
                  pl.BlockSpec((1, D), lambda q: (0, 0)),
                  pl.BlockSpec((1, R, BUCKET, D), lambda q: (q, 0, 0, 0)),
                  pl.BlockSpec((1, R, BUCKET, 1), lambda q: (q, 0, 0, 0)),
                  pl.BlockSpec((R, D, D), lambda q: (0, 0, 0))],
        out_specs=pl.BlockSpec((BUCKET, D), lambda q: (q, 0)),
        out_shape=jax.ShapeDtypeStruct((N, D), jnp.float32),
    )(h, root, b, s, cnt4, w)


def kernel(x, edge_index, edge_type, lin_w, lin_b, comp1, bases1, root1,
           bias1, comp2, bases2, root2, bias2):
    src = edge_index[0]
    dst = edge_index[1]
    h0 = _tc_linear(x, lin_w, lin_b.reshape(1, D))
    w1 = _tc_basis(comp1, bases1)
    w2 = _tc_basis(comp2, bases2)
    s1, cnt = _make_sc_segsum(True)(h0, src, dst, edge_type)
    cnt4 = cnt.reshape(NBKT, R, BUCKET, 1)
    h1 = _tc_combine(h0, root1, bias1.reshape(1, D), s1, cnt4, w1, False)
    (s2,) = _make_sc_segsum(False)(h1, src, dst, edge_type)
    h2 = _tc_combine(h1, root2, bias2.reshape(1, D), s2, cnt4, w2, True)
    return h2
